# baseline (speedup 1.0000x reference)
import functools

import jax
import jax.numpy as jnp
from jax import lax
from jax.experimental import pallas as pl
from jax.experimental.pallas import tpu as pltpu
from jax.experimental.pallas import tpu_sc as plsc

N_NODES = 100000
C = 20
M = 128
N_GEN = 16
_TAB = N_GEN * M

_NC = 2
_NS = 16
_NW = _NC * _NS
_B_PER_W = 3136
_B_LAST = N_NODES - (_NW - 1) * _B_PER_W


def _table_body(bt_ref, pi_ref, t_ref):
    bt = bt_ref[...]
    pi = pi_ref[...]
    sm_b = jax.nn.softmax(bt, axis=2)
    sm_pi = jax.nn.softmax(pi, axis=0)
    acc = jnp.sum(sm_pi[:, :, None] * sm_b, axis=0)
    t_ref[...] = jnp.log(acc + C * 1e-12)


def _compute_table_gm(B, Pi):
    return pl.pallas_call(
        _table_body,
        out_shape=jax.ShapeDtypeStruct((N_GEN, M), jnp.float32),
    )(jnp.transpose(B, (0, 2, 1)), Pi)


_MESH = plsc.VectorSubcoreMesh(core_axis_name="c", subcore_axis_name="s")


@functools.partial(
    pl.kernel,
    mesh=_MESH,
    out_type=jax.ShapeDtypeStruct((N_NODES, 1, N_GEN), jnp.float32),
    scratch_types=[
        pltpu.VMEM((_TAB,), jnp.float32),
        pltpu.VMEM((_B_PER_W,), jnp.int32),
        pltpu.VMEM((_B_PER_W, 1, N_GEN), jnp.float32),
    ],
    compiler_params=pltpu.CompilerParams(
        use_tc_tiling_on_sc=False, needs_layout_passes=False),
)
def _lookup_kernel(idx_hbm, tab_hbm, out_hbm, tab_v, idx_v, rows_v):
    wid = lax.axis_index("s") * _NC + lax.axis_index("c")
    base = wid * _B_PER_W
    lane = lax.iota(jnp.int32, 16)
    zero16 = jnp.zeros((16,), jnp.int32)

    pltpu.sync_copy(tab_hbm, tab_v)

    def _chunk(n):
        pltpu.sync_copy(idx_hbm.at[pl.ds(base, n)], idx_v.at[pl.ds(0, n)])

        def body(i, carry):
            row = i * 16 + lane
            xv = plsc.load_gather(idx_v, [row])
            for k in range(N_GEN):
                col = (k + lane) & 15
                vals = plsc.load_gather(tab_v, [col * M + xv])
                plsc.store_scatter(rows_v, [row, zero16, col], vals)
            return carry

        lax.fori_loop(0, n // 16, body, 0)
        pltpu.sync_copy(rows_v.at[pl.ds(0, n)], out_hbm.at[pl.ds(base, n)])

    @pl.when(wid < _NW - 1)
    def _full_chunk():
        _chunk(_B_PER_W)

    @pl.when(wid == _NW - 1)
    def _tail_chunk():
        _chunk(_B_LAST)


def kernel(x, edge_index, B, Pi):
    del edge_index
    tab_flat = _compute_table_gm(B, Pi).reshape(_TAB)
    return _lookup_kernel(x, tab_flat)


# speedup vs baseline: 1.5648x; 1.5648x over previous
import functools

import jax
import jax.numpy as jnp
from jax import lax
from jax.experimental import pallas as pl
from jax.experimental.pallas import tpu as pltpu
from jax.experimental.pallas import tpu_sc as plsc

N_NODES = 100000
C = 20
M = 128
N_GEN = 16
_TAB = N_GEN * M

_NC = 2
_NS = 16
_NW = _NC * _NS
_B_PER_W = 3136
_B_LAST = N_NODES - (_NW - 1) * _B_PER_W


def _table_body(b_ref, pi_ref, t_ref):
    b = b_ref[...]
    pi = pi_ref[...]
    sm_b = jax.nn.softmax(b, axis=1)
    sm_pi = jax.nn.softmax(pi, axis=0)
    acc = jnp.sum(sm_pi[:, None, :] * sm_b, axis=0)
    t_ref[...] = jnp.log(acc + C * 1e-12)


def _compute_table_mg(B, Pi):
    return pl.pallas_call(
        _table_body,
        out_shape=jax.ShapeDtypeStruct((M, N_GEN), jnp.float32),
    )(B, Pi)


_MESH = plsc.VectorSubcoreMesh(core_axis_name="c", subcore_axis_name="s")


@functools.partial(
    pl.kernel,
    mesh=_MESH,
    out_type=jax.ShapeDtypeStruct((N_NODES, N_GEN), jnp.float32),
    scratch_types=[
        pltpu.VMEM((_B_PER_W,), jnp.int32),
        pltpu.VMEM((_B_PER_W, N_GEN), jnp.float32),
        pltpu.SemaphoreType.DMA,
    ],
    compiler_params=pltpu.CompilerParams(
        use_tc_tiling_on_sc=False, needs_layout_passes=False),
)
def _lookup_kernel(idx_hbm, table_hbm, out_hbm, idx_v, rows_v, sem):
    wid = lax.axis_index("s") * _NC + lax.axis_index("c")
    base = wid * _B_PER_W

    def _chunk(n):
        pltpu.sync_copy(idx_hbm.at[pl.ds(base, n)], idx_v.at[pl.ds(0, n)])
        pltpu.async_copy(table_hbm.at[idx_v.at[pl.ds(0, n)]],
                         rows_v.at[pl.ds(0, n)], sem).wait()
        pltpu.sync_copy(rows_v.at[pl.ds(0, n)], out_hbm.at[pl.ds(base, n)])

    @pl.when(wid < _NW - 1)
    def _full_chunk():
        _chunk(_B_PER_W)

    @pl.when(wid == _NW - 1)
    def _tail_chunk():
        _chunk(_B_LAST)


def kernel(x, edge_index, B, Pi):
    del edge_index
    table = _compute_table_mg(B, Pi)
    rows = _lookup_kernel(x, table)
    scale = 1.0 + 0.0 * Pi[0, 0]
    return rows[:, None, :] * scale


# speedup vs baseline: 2.7898x; 1.7829x over previous
import functools

import jax
import jax.numpy as jnp
from jax import lax
from jax.experimental import pallas as pl
from jax.experimental.pallas import tpu as pltpu
from jax.experimental.pallas import tpu_sc as plsc

N_NODES = 100000
C = 20
M = 128
N_GEN = 16
_TAB = N_GEN * M

_NC = 2
_NS = 16
_NW = _NC * _NS
_B_PER_W = 3136
_B_LAST = N_NODES - (_NW - 1) * _B_PER_W


def _table_body(bt_ref, pi_ref, t_ref):
    bt = bt_ref[...]
    pi = pi_ref[...]
    sm_b = jax.nn.softmax(bt, axis=2)
    sm_pi = jax.nn.softmax(pi, axis=0)
    acc = jnp.sum(sm_pi[:, :, None] * sm_b, axis=0)
    t_ref[...] = jnp.log(acc + C * 1e-12)


def _compute_table_gm(B, Pi):
    return pl.pallas_call(
        _table_body,
        out_shape=jax.ShapeDtypeStruct((N_GEN, M), jnp.float32),
    )(jnp.transpose(B, (0, 2, 1)), Pi)


_MESH = plsc.VectorSubcoreMesh(core_axis_name="c", subcore_axis_name="s")


@functools.partial(
    pl.kernel,
    mesh=_MESH,
    out_type=jax.ShapeDtypeStruct((N_NODES, N_GEN), jnp.float32),
    scratch_types=[
        pltpu.VMEM((_TAB,), jnp.float32),
        pltpu.VMEM((_B_PER_W,), jnp.int32),
        pltpu.VMEM((_B_PER_W, N_GEN), jnp.float32),
    ],
    compiler_params=pltpu.CompilerParams(
        use_tc_tiling_on_sc=False, needs_layout_passes=False),
)
def _lookup_kernel(idx_hbm, tab_hbm, out_hbm, tab_v, idx_v, rows_v):
    wid = lax.axis_index("s") * _NC + lax.axis_index("c")
    base = wid * _B_PER_W
    lane = lax.iota(jnp.int32, 16)

    pltpu.sync_copy(tab_hbm, tab_v)

    def _chunk(n):
        pltpu.sync_copy(idx_hbm.at[pl.ds(base, n)], idx_v.at[pl.ds(0, n)])

        @plsc.parallel_loop(0, n // 16, 1, unroll=8)
        def _batch(i):
            row = i * 16 + lane
            xv = plsc.load_gather(idx_v, [row])
            for k in range(N_GEN):
                col = (k + lane) & 15
                vals = plsc.load_gather(tab_v, [col * M + xv])
                plsc.store_scatter(rows_v, [row, col], vals)

        pltpu.sync_copy(rows_v.at[pl.ds(0, n)], out_hbm.at[pl.ds(base, n)])

    @pl.when(wid < _NW - 1)
    def _full_chunk():
        _chunk(_B_PER_W)

    @pl.when(wid == _NW - 1)
    def _tail_chunk():
        _chunk(_B_LAST)


def kernel(x, edge_index, B, Pi):
    del edge_index
    tab_flat = _compute_table_gm(B, Pi).reshape(_TAB)
    rows = _lookup_kernel(x, tab_flat)
    return rows[:, None, :]
